# Initial kernel scaffold; baseline (speedup 1.0000x reference)
import functools

import jax
import jax.numpy as jnp
from jax import lax
from jax.experimental import pallas as pl
from jax.experimental.pallas import tpu as pltpu
from jax.experimental.pallas import tpu_sc as plsc

D = 128
NX = 20000
NB = 10000
HALF = 10000
NTILE = 16
SROWS = HALF + 16
STEP = 80

_DN = (((1,), (1,)), ((), ()))
_HP = lax.Precision.HIGHEST


def _mm3_body(x_ref, wa_ref, wb_ref, ws_ref, bm_ref, bs_ref,
              t1_ref, y2_ref, so_ref):
    x = x_ref[...]
    t1_ref[...] = lax.dot_general(x, wa_ref[...], _DN, precision=_HP,
                                  preferred_element_type=jnp.float32)
    y2_ref[...] = lax.dot_general(x, wb_ref[...], _DN, precision=_HP,
                                  preferred_element_type=jnp.float32) + bm_ref[...]
    so_ref[...] = lax.dot_general(x, ws_ref[...], _DN, precision=_HP,
                                  preferred_element_type=jnp.float32) + bs_ref[...]


def _mm3(x, wa, wb, ws, bm, bs, rblk):
    n = x.shape[0]
    f = pl.pallas_call(
        _mm3_body,
        grid=(n // rblk,),
        in_specs=[
            pl.BlockSpec((rblk, D), lambda i: (i, 0)),
            pl.BlockSpec((D, D), lambda i: (0, 0)),
            pl.BlockSpec((D, D), lambda i: (0, 0)),
            pl.BlockSpec((D, D), lambda i: (0, 0)),
            pl.BlockSpec((1, D), lambda i: (0, 0)),
            pl.BlockSpec((1, D), lambda i: (0, 0)),
        ],
        out_specs=[pl.BlockSpec((rblk, D), lambda i: (i, 0))] * 3,
        out_shape=[jax.ShapeDtypeStruct((n, D), jnp.float32)] * 3,
    )
    return f(x, wa, wb, ws, bm, bs)


def _final_body(s_ref, c_ref, y2_ref, so_ref, wu_ref, bu_ref, o_ref):
    cnt = c_ref[...][:, 0:1]
    agg = (s_ref[...] + cnt * y2_ref[...]) / jnp.maximum(cnt, 1.0)
    o_ref[...] = so_ref[...] + lax.dot_general(
        agg, wu_ref[...], _DN, precision=_HP,
        preferred_element_type=jnp.float32) + bu_ref[...]


def _final(S, C, y2, so, wu, bu, rblk):
    n = S.shape[0]
    f = pl.pallas_call(
        _final_body,
        grid=(n // rblk,),
        in_specs=[
            pl.BlockSpec((rblk, D), lambda i: (i, 0)),
            pl.BlockSpec((rblk, 16), lambda i: (i, 0)),
            pl.BlockSpec((rblk, D), lambda i: (i, 0)),
            pl.BlockSpec((rblk, D), lambda i: (i, 0)),
            pl.BlockSpec((D, D), lambda i: (0, 0)),
            pl.BlockSpec((1, D), lambda i: (0, 0)),
        ],
        out_specs=pl.BlockSpec((rblk, D), lambda i: (i, 0)),
        out_shape=jax.ShapeDtypeStruct((n, D), jnp.float32),
    )
    return f(S, C, y2, so, wu, bu)


def _sc_scatter(t1x, t1b, si, di, sb, db):
    ei = si.shape[0]
    eb = sb.shape[0]
    ci = ei // NTILE
    cb = eb // NTILE
    nsi = ci // STEP
    nsb = cb // STEP
    nz = SROWS // NTILE
    nf = HALF // NTILE

    mesh = plsc.VectorSubcoreMesh(core_axis_name="c", subcore_axis_name="s")

    @functools.partial(
        pl.kernel,
        mesh=mesh,
        out_type=[
            jax.ShapeDtypeStruct((2 * HALF, D), jnp.float32),
            jax.ShapeDtypeStruct((2 * HALF, 16), jnp.float32),
        ],
        scratch_types=[
            pltpu.VMEM((ci,), jnp.int32),
            pltpu.VMEM((ci,), jnp.int32),
            pltpu.VMEM((cb,), jnp.int32),
            pltpu.VMEM((cb,), jnp.int32),
            pltpu.VMEM((2, STEP), jnp.int32),
            pltpu.VMEM((2, STEP), jnp.int32),
            pltpu.VMEM((2, STEP, D), jnp.float32),
            pltpu.VMEM((STEP, 16), jnp.float32),
            pltpu.VMEM((64, D), jnp.float32),
            pltpu.VMEM((64, 16), jnp.float32),
            pltpu.VMEM_SHARED((SROWS, D), jnp.float32),
            pltpu.VMEM_SHARED((SROWS, 16), jnp.float32),
            pltpu.SemaphoreType.DMA,
            pltpu.SemaphoreType.DMA,
        ],
    )
    def k(t1x_h, t1b_h, si_h, di_h, sb_h, db_h, s_out, c_out,
          si_v, di_v, sb_v, db_v, gidx_v, sidx_v, rows_v, ones_v, zb, zbc,
          s_sh, c_sh, sem0, sem1):
        c = lax.axis_index("c")
        s = lax.axis_index("s")
        base = c * HALF

        pltpu.sync_copy(si_h.at[pl.ds(s * ci, ci)], si_v)
        pltpu.sync_copy(di_h.at[pl.ds(s * ci, ci)], di_v)
        pltpu.sync_copy(sb_h.at[pl.ds(s * cb, cb)], sb_v)
        pltpu.sync_copy(db_h.at[pl.ds(s * cb, cb)], db_v)

        zero16 = jnp.zeros((16,), jnp.float32)
        one16 = jnp.ones((16,), jnp.float32)

        def zrow(r, carry):
            for kk in range(D // 16):
                zb[r, pl.ds(kk * 16, 16)] = zero16
            zbc[r, :] = zero16
            return carry

        lax.fori_loop(0, 64, zrow, 0)

        def orow(r, carry):
            ones_v[r, :] = one16
            return carry

        lax.fori_loop(0, STEP, orow, 0)

        r0 = s * nz
        for kk in range(nz // 64):
            pltpu.sync_copy(zb, s_sh.at[pl.ds(r0 + kk * 64, 64)])
            pltpu.sync_copy(zbc, c_sh.at[pl.ds(r0 + kk * 64, 64)])
        rem = nz % 64
        if rem:
            pltpu.sync_copy(zb.at[pl.ds(0, rem)],
                            s_sh.at[pl.ds(r0 + (nz // 64) * 64, rem)])
            pltpu.sync_copy(zbc.at[pl.ds(0, rem)],
                            c_sh.at[pl.ds(r0 + (nz // 64) * 64, rem)])

        plsc.subcore_barrier()

        def phase(table, src_v, dst_v, nsteps, is_bound):
            def gidx_for(j, p):
                for kk in range(STEP // 16):
                    v = src_v[pl.ds(j * STEP + kk * 16, 16)]
                    if is_bound:
                        keep = (v >= NB) & (v < NX)
                        g = jnp.where(keep, v - NB, 0)
                    else:
                        g = v
                    gidx_v[p, pl.ds(kk * 16, 16)] = g

            def sidx_for(j, p):
                for kk in range(STEP // 16):
                    dd = dst_v[pl.ds(j * STEP + kk * 16, 16)]
                    loc = dd - base
                    ok = (loc >= 0) & (loc < HALF)
                    if is_bound:
                        v = src_v[pl.ds(j * STEP + kk * 16, 16)]
                        ok = ok & (v >= NB) & (v < NX)
                    sidx_v[p, pl.ds(kk * 16, 16)] = jnp.where(ok, loc, HALF)

            def start(j, p):
                gidx_for(j, p)
                sem = sem1 if p else sem0
                pltpu.async_copy(table.at[gidx_v.at[p]], rows_v.at[p], sem)

            def wait(p):
                sem = sem1 if p else sem0
                pltpu.make_async_copy(table.at[gidx_v.at[p]], rows_v.at[p],
                                      sem).wait()

            start(0, 0)
            start(1, 1)
            npair = nsteps // 2

            def pair(g, carry):
                for p in range(2):
                    j = g * 2 + p
                    wait(p)
                    sidx_for(j, p)
                    pltpu.sync_copy(rows_v.at[p], s_sh.at[sidx_v.at[p]],
                                    add=True)
                    pltpu.sync_copy(ones_v, c_sh.at[sidx_v.at[p]], add=True)

                    @pl.when(g + 1 < npair)
                    def _():
                        start(j + 2, p)

                return carry

            lax.fori_loop(0, npair, pair, 0)

        phase(t1x_h, si_v, di_v, nsi, False)
        phase(t1b_h, sb_v, db_v, nsb, True)

        plsc.subcore_barrier()

        f0 = s * nf
        pltpu.sync_copy(s_sh.at[pl.ds(f0, nf)], s_out.at[pl.ds(base + f0, nf)])
        pltpu.sync_copy(c_sh.at[pl.ds(f0, nf)], c_out.at[pl.ds(base + f0, nf)])

    return k(t1x, t1b, si, di, sb, db)


def kernel(x_int, bv, edge_index_int, edge_index_bound,
           W_msg, b_msg, W_self, b_self, W_upd, b_upd):
    x = x_int.reshape(-1, D)
    b = bv.reshape(-1, D)
    wa = W_msg[:, :D]
    wb = W_msg[:, D:]
    bm = b_msg.reshape(1, D)
    bs = b_self.reshape(1, D)
    bu = b_upd.reshape(1, D)

    t1x, y2pb, selfx = _mm3(x, wa, wb, W_self, bm, bs, 2000)
    t1b, _, selfb = _mm3(b, wa, wb, W_self, bm, bs, 2000)

    si = edge_index_int[0].astype(jnp.int32)
    di = edge_index_int[1].astype(jnp.int32)
    sb = edge_index_bound[0].astype(jnp.int32)
    db = edge_index_bound[1].astype(jnp.int32)

    S, C = _sc_scatter(t1x, t1b, si, di, sb, db)
    xu = _final(S, C, y2pb, selfx, W_upd, bu, 2000)
    return xu[None, ...], selfb[None, ...]


# speedup vs baseline: 1.3727x; 1.3727x over previous
import functools

import jax
import jax.numpy as jnp
from jax import lax
from jax.experimental import pallas as pl
from jax.experimental.pallas import tpu as pltpu
from jax.experimental.pallas import tpu_sc as plsc

D = 128
NX = 20000
NB = 10000
HALF = 10000
NTILE = 16
SROWS = 10112
STEP = 80

_DN = (((1,), (1,)), ((), ()))
_HP = lax.Precision.HIGHEST


def _mm3_body(x_ref, wa_ref, wb_ref, ws_ref, bm_ref, bs_ref,
              t1_ref, y2_ref, so_ref):
    x = x_ref[...]
    t1_ref[...] = lax.dot_general(x, wa_ref[...], _DN, precision=_HP,
                                  preferred_element_type=jnp.float32)
    y2_ref[...] = lax.dot_general(x, wb_ref[...], _DN, precision=_HP,
                                  preferred_element_type=jnp.float32) + bm_ref[...]
    so_ref[...] = lax.dot_general(x, ws_ref[...], _DN, precision=_HP,
                                  preferred_element_type=jnp.float32) + bs_ref[...]


def _mm3(x, wa, wb, ws, bm, bs, rblk):
    n = x.shape[0]
    f = pl.pallas_call(
        _mm3_body,
        grid=(n // rblk,),
        in_specs=[
            pl.BlockSpec((rblk, D), lambda i: (i, 0)),
            pl.BlockSpec((D, D), lambda i: (0, 0)),
            pl.BlockSpec((D, D), lambda i: (0, 0)),
            pl.BlockSpec((D, D), lambda i: (0, 0)),
            pl.BlockSpec((1, D), lambda i: (0, 0)),
            pl.BlockSpec((1, D), lambda i: (0, 0)),
        ],
        out_specs=[pl.BlockSpec((rblk, D), lambda i: (i, 0))] * 3,
        out_shape=[jax.ShapeDtypeStruct((n, D), jnp.float32)] * 3,
    )
    return f(x, wa, wb, ws, bm, bs)


def _final_body(s_ref, c_ref, y2_ref, so_ref, wu_ref, bu_ref, o_ref):
    cnt = c_ref[...].reshape(-1, 1)
    agg = (s_ref[...] + cnt * y2_ref[...]) / jnp.maximum(cnt, 1.0)
    o_ref[...] = so_ref[...] + lax.dot_general(
        agg, wu_ref[...], _DN, precision=_HP,
        preferred_element_type=jnp.float32) + bu_ref[...]


def _final(S, C, y2, so, wu, bu, rblk):
    n = S.shape[0]
    f = pl.pallas_call(
        _final_body,
        grid=(n // rblk,),
        in_specs=[
            pl.BlockSpec((rblk, D), lambda i: (i, 0)),
            pl.BlockSpec((1, 1, rblk), lambda i: (i, 0, 0)),
            pl.BlockSpec((rblk, D), lambda i: (i, 0)),
            pl.BlockSpec((rblk, D), lambda i: (i, 0)),
            pl.BlockSpec((D, D), lambda i: (0, 0)),
            pl.BlockSpec((1, D), lambda i: (0, 0)),
        ],
        out_specs=pl.BlockSpec((rblk, D), lambda i: (i, 0)),
        out_shape=jax.ShapeDtypeStruct((n, D), jnp.float32),
    )
    return f(S, C, y2, so, wu, bu)


def _sc_scatter(t1x, t1b, si, di, sb, db):
    ei = si.shape[0]
    eb = sb.shape[0]
    ci = ei // NTILE
    cb = eb // NTILE
    SB = 800
    nz = SROWS // NTILE
    nf = 624

    mesh = plsc.VectorSubcoreMesh(core_axis_name="c", subcore_axis_name="s")

    @functools.partial(
        pl.kernel,
        mesh=mesh,
        out_type=[
            jax.ShapeDtypeStruct((2 * HALF, D), jnp.float32),
            jax.ShapeDtypeStruct((2 * HALF,), jnp.float32),
        ],
        scratch_types=[
            pltpu.VMEM((SB,), jnp.int32),
            pltpu.VMEM((SB,), jnp.int32),
            pltpu.VMEM((STEP,), jnp.int32),
            pltpu.VMEM((STEP,), jnp.int32),
            pltpu.VMEM((STEP,), jnp.int32),
            pltpu.VMEM((STEP,), jnp.int32),
            pltpu.VMEM((2, STEP, D), jnp.float32),
            pltpu.VMEM((STEP,), jnp.float32),
            pltpu.VMEM((8, D), jnp.float32),
            pltpu.VMEM((640,), jnp.float32),
            pltpu.VMEM_SHARED((SROWS, D), jnp.float32),
            pltpu.VMEM_SHARED((SROWS,), jnp.float32),
            pltpu.SemaphoreType.DMA,
            pltpu.SemaphoreType.DMA,
        ],
    )
    def k(t1x_h, t1b_h, si_h, di_h, sb_h, db_h, s_out, c_out,
          si_v, di_v, gidx0, gidx1, sidx0, sidx1, rows_v, ones_v, zb, zc,
          s_sh, c_sh, sem0, sem1):
        c = lax.axis_index("c")
        s = lax.axis_index("s")
        base = c * HALF

        zero16 = jnp.zeros((16,), jnp.float32)
        one16 = jnp.ones((16,), jnp.float32)

        def zrow(r, carry):
            for kk in range(D // 16):
                zb[r, pl.ds(kk * 16, 16)] = zero16
            return carry

        lax.fori_loop(0, 8, zrow, 0)

        def zrow1(r, carry):
            zc[pl.ds(r * 16, 16)] = zero16
            return carry

        lax.fori_loop(0, 40, zrow1, 0)

        def orow(r, carry):
            ones_v[pl.ds(r * 16, 16)] = one16
            return carry

        lax.fori_loop(0, STEP // 16, orow, 0)

        r0 = s * nz
        for kk in range(nz // 8):
            pltpu.sync_copy(zb, s_sh.at[pl.ds(r0 + kk * 8, 8)])
        pltpu.sync_copy(zc.at[pl.ds(0, nz)], c_sh.at[pl.ds(r0, nz)])

        plsc.subcore_barrier()

        NS = SB // STEP
        gbufs = (gidx0, gidx1)
        sbufs = (sidx0, sidx1)

        def phase(table, src_h, dst_h, chunk, is_bound):
            def gidx_for(j, p):
                for kk in range(STEP // 16):
                    v = si_v[pl.ds(j * STEP + kk * 16, 16)]
                    if is_bound:
                        keep = (v >= NB) & (v < NX)
                        g = jnp.where(keep, v - NB, 0)
                    else:
                        g = v
                    gbufs[p][pl.ds(kk * 16, 16)] = g

            def sidx_for(j, p):
                for kk in range(STEP // 16):
                    dd = di_v[pl.ds(j * STEP + kk * 16, 16)]
                    loc = dd - base
                    ok = (loc >= 0) & (loc < HALF)
                    if is_bound:
                        v = si_v[pl.ds(j * STEP + kk * 16, 16)]
                        ok = ok & (v >= NB) & (v < NX)
                    sbufs[p][pl.ds(kk * 16, 16)] = jnp.where(ok, loc, HALF)

            def superblock(blk, carry):
                off = s * chunk + blk * SB
                pltpu.sync_copy(src_h.at[pl.ds(off, SB)], si_v)
                pltpu.sync_copy(dst_h.at[pl.ds(off, SB)], di_v)

                def step(j, inner):
                    gidx_for(j, 0)
                    h = pltpu.async_copy(table.at[gidx0], rows_v.at[0], sem0)
                    sidx_for(j, 0)
                    h.wait()
                    pltpu.sync_copy(rows_v.at[0], s_sh.at[sidx0], add=True)
                    pltpu.sync_copy(ones_v, c_sh.at[sidx0], add=True)
                    return inner

                lax.fori_loop(0, NS, step, 0)
                return carry

            lax.fori_loop(0, chunk // SB, superblock, 0)

        phase(t1x_h, si_h, di_h, ci, False)
        phase(t1b_h, sb_h, db_h, cb, True)

        plsc.subcore_barrier()

        f0 = s * nf
        pltpu.sync_copy(s_sh.at[pl.ds(f0, nf)], s_out.at[pl.ds(base + f0, nf)])
        pltpu.sync_copy(c_sh.at[pl.ds(f0, nf)], zc.at[pl.ds(0, 624)])
        pltpu.sync_copy(zc.at[pl.ds(0, 624)], c_out.at[pl.ds(base + f0, 624)])
        tail = NTILE * nf
        trem = HALF - tail

        @pl.when(s == 0)
        def _():
            pltpu.sync_copy(s_sh.at[pl.ds(tail, trem)],
                            s_out.at[pl.ds(base + tail, trem)])
            pltpu.sync_copy(c_sh.at[pl.ds(tail, trem)], zc.at[pl.ds(624, trem)])
            pltpu.sync_copy(zc.at[pl.ds(624, trem)],
                            c_out.at[pl.ds(base + tail, trem)])

    return k(t1x, t1b, si, di, sb, db)


def kernel(x_int, bv, edge_index_int, edge_index_bound,
           W_msg, b_msg, W_self, b_self, W_upd, b_upd):
    x = x_int.reshape(-1, D)
    b = bv.reshape(-1, D)
    wa = W_msg[:, :D]
    wb = W_msg[:, D:]
    bm = b_msg.reshape(1, D)
    bs = b_self.reshape(1, D)
    bu = b_upd.reshape(1, D)

    t1x, y2pb, selfx = _mm3(x, wa, wb, W_self, bm, bs, 2000)
    t1b, _, selfb = _mm3(b, wa, wb, W_self, bm, bs, 2000)

    si = edge_index_int[0].astype(jnp.int32)
    di = edge_index_int[1].astype(jnp.int32)
    sb = edge_index_bound[0].astype(jnp.int32)
    db = edge_index_bound[1].astype(jnp.int32)

    S, C = _sc_scatter(t1x, t1b, si, di, sb, db)
    xu = _final(S, C.reshape(10, 1, 2000), y2pb, selfx, W_upd, bu, 2000)
    return xu[None, ...], selfb[None, ...]


# speedup vs baseline: 1.4331x; 1.0440x over previous
import functools

import jax
import jax.numpy as jnp
from jax import lax
from jax.experimental import pallas as pl
from jax.experimental.pallas import tpu as pltpu
from jax.experimental.pallas import tpu_sc as plsc

D = 128
NX = 20000
NB = 10000
HALF = 10000
NTILE = 16
SROWS = 10112
STEP = 80

_DN = (((1,), (1,)), ((), ()))
_HP = lax.Precision.HIGHEST


def _mm3_body(x_ref, wa_ref, wb_ref, ws_ref, bm_ref, bs_ref,
              t1_ref, y2_ref, so_ref):
    x = x_ref[...]
    t1_ref[...] = lax.dot_general(x, wa_ref[...], _DN, precision=_HP,
                                  preferred_element_type=jnp.float32)
    y2_ref[...] = lax.dot_general(x, wb_ref[...], _DN, precision=_HP,
                                  preferred_element_type=jnp.float32) + bm_ref[...]
    so_ref[...] = lax.dot_general(x, ws_ref[...], _DN, precision=_HP,
                                  preferred_element_type=jnp.float32) + bs_ref[...]


def _mm3(x, wa, wb, ws, bm, bs, rblk):
    n = x.shape[0]
    f = pl.pallas_call(
        _mm3_body,
        grid=(n // rblk,),
        in_specs=[
            pl.BlockSpec((rblk, D), lambda i: (i, 0)),
            pl.BlockSpec((D, D), lambda i: (0, 0)),
            pl.BlockSpec((D, D), lambda i: (0, 0)),
            pl.BlockSpec((D, D), lambda i: (0, 0)),
            pl.BlockSpec((1, D), lambda i: (0, 0)),
            pl.BlockSpec((1, D), lambda i: (0, 0)),
        ],
        out_specs=[pl.BlockSpec((rblk, D), lambda i: (i, 0))] * 3,
        out_shape=[jax.ShapeDtypeStruct((n, D), jnp.float32)] * 3,
    )
    return f(x, wa, wb, ws, bm, bs)


def _final_body(s_ref, c_ref, y2_ref, so_ref, wu_ref, bu_ref, o_ref):
    cnt = c_ref[...].reshape(-1, 1)
    agg = (s_ref[...] + cnt * y2_ref[...]) / jnp.maximum(cnt, 1.0)
    o_ref[...] = so_ref[...] + lax.dot_general(
        agg, wu_ref[...], _DN, precision=_HP,
        preferred_element_type=jnp.float32) + bu_ref[...]


def _final(S, C, y2, so, wu, bu, rblk):
    n = S.shape[0]
    f = pl.pallas_call(
        _final_body,
        grid=(n // rblk,),
        in_specs=[
            pl.BlockSpec((rblk, D), lambda i: (i, 0)),
            pl.BlockSpec((1, 1, rblk), lambda i: (i, 0, 0)),
            pl.BlockSpec((rblk, D), lambda i: (i, 0)),
            pl.BlockSpec((rblk, D), lambda i: (i, 0)),
            pl.BlockSpec((D, D), lambda i: (0, 0)),
            pl.BlockSpec((1, D), lambda i: (0, 0)),
        ],
        out_specs=pl.BlockSpec((rblk, D), lambda i: (i, 0)),
        out_shape=jax.ShapeDtypeStruct((n, D), jnp.float32),
    )
    return f(S, C, y2, so, wu, bu)


def _sc_scatter(t1x, t1b, si, di, sb, db):
    ei = si.shape[0]
    eb = sb.shape[0]
    ci = ei // NTILE
    cb = eb // NTILE
    SB = 4000
    nz = SROWS // NTILE
    nf = 624

    mesh = plsc.VectorSubcoreMesh(core_axis_name="c", subcore_axis_name="s")

    @functools.partial(
        pl.kernel,
        mesh=mesh,
        out_type=[
            jax.ShapeDtypeStruct((2 * HALF, D), jnp.float32),
            jax.ShapeDtypeStruct((2 * HALF,), jnp.float32),
        ],
        scratch_types=[
            pltpu.VMEM((SB,), jnp.int32),
            pltpu.VMEM((SB,), jnp.int32),
            pltpu.VMEM((STEP,), jnp.int32),
            pltpu.VMEM((STEP,), jnp.int32),
            pltpu.VMEM((STEP,), jnp.int32),
            pltpu.VMEM((STEP,), jnp.int32),
            pltpu.VMEM((2, STEP, D), jnp.float32),
            pltpu.VMEM((STEP,), jnp.float32),
            pltpu.VMEM((8, D), jnp.float32),
            pltpu.VMEM((640,), jnp.float32),
            pltpu.VMEM_SHARED((SROWS, D), jnp.float32),
            pltpu.VMEM_SHARED((SROWS,), jnp.float32),
            pltpu.SemaphoreType.DMA,
            pltpu.SemaphoreType.DMA,
            pltpu.SemaphoreType.DMA,
            pltpu.SemaphoreType.DMA,
            pltpu.SemaphoreType.DMA,
            pltpu.SemaphoreType.DMA,
        ],
    )
    def k(t1x_h, t1b_h, si_h, di_h, sb_h, db_h, s_out, c_out,
          si_v, di_v, gidx0, gidx1, sidx0, sidx1, rows_v, ones_v, zb, zc,
          s_sh, c_sh, semg0, semg1, sems0, sems1, semo0, semo1):
        c = lax.axis_index("c")
        s = lax.axis_index("s")
        base = c * HALF

        zero16 = jnp.zeros((16,), jnp.float32)
        one16 = jnp.ones((16,), jnp.float32)

        def zrow(r, carry):
            for kk in range(D // 16):
                zb[r, pl.ds(kk * 16, 16)] = zero16
            return carry

        lax.fori_loop(0, 8, zrow, 0)

        def zrow1(r, carry):
            zc[pl.ds(r * 16, 16)] = zero16
            return carry

        lax.fori_loop(0, 40, zrow1, 0)

        def orow(r, carry):
            ones_v[pl.ds(r * 16, 16)] = one16
            return carry

        lax.fori_loop(0, STEP // 16, orow, 0)

        r0 = s * nz
        for kk in range(nz // 8):
            pltpu.sync_copy(zb, s_sh.at[pl.ds(r0 + kk * 8, 8)])
        pltpu.sync_copy(zc.at[pl.ds(0, nz)], c_sh.at[pl.ds(r0, nz)])

        plsc.subcore_barrier()

        NS = SB // STEP
        npair = NS // 2
        gbufs = (gidx0, gidx1)
        sbufs = (sidx0, sidx1)
        semg = (semg0, semg1)
        sems = (sems0, sems1)
        semo = (semo0, semo1)

        def phase(table, src_h, dst_h, chunk, is_bound):
            def gidx_for(j, p):
                for kk in range(STEP // 16):
                    v = si_v[pl.ds(j * STEP + kk * 16, 16)]
                    if is_bound:
                        keep = (v >= NB) & (v < NX)
                        g = jnp.where(keep, v - NB, 0)
                    else:
                        g = v
                    gbufs[p][pl.ds(kk * 16, 16)] = g

            def sidx_for(j, p):
                for kk in range(STEP // 16):
                    dd = di_v[pl.ds(j * STEP + kk * 16, 16)]
                    loc = dd - base
                    ok = (loc >= 0) & (loc < HALF)
                    if is_bound:
                        v = si_v[pl.ds(j * STEP + kk * 16, 16)]
                        ok = ok & (v >= NB) & (v < NX)
                    sbufs[p][pl.ds(kk * 16, 16)] = jnp.where(ok, loc, HALF)

            def start_gather(p):
                pltpu.async_copy(table.at[gbufs[p]], rows_v.at[p], semg[p])

            def wait_gather(p):
                pltpu.make_async_copy(table.at[gbufs[p]], rows_v.at[p],
                                      semg[p]).wait()

            def start_scatter(p):
                pltpu.async_copy(rows_v.at[p], s_sh.at[sbufs[p]], sems[p],
                                 add=True)
                pltpu.async_copy(ones_v, c_sh.at[sbufs[p]], semo[p], add=True)

            def wait_scatter(p):
                pltpu.make_async_copy(rows_v.at[p], s_sh.at[sbufs[p]],
                                      sems[p]).wait()
                pltpu.make_async_copy(ones_v, c_sh.at[sbufs[p]],
                                      semo[p]).wait()

            def superblock(blk, carry):
                off = s * chunk + blk * SB
                pltpu.sync_copy(src_h.at[pl.ds(off, SB)], si_v)
                pltpu.sync_copy(dst_h.at[pl.ds(off, SB)], di_v)

                gidx_for(0, 0)
                start_gather(0)

                def pair(g, inner):
                    j = 2 * g
                    wait_gather(0)

                    @pl.when(g > 0)
                    def _():
                        wait_scatter(1)

                    sidx_for(j, 0)
                    start_scatter(0)
                    gidx_for(j + 1, 1)
                    start_gather(1)

                    wait_gather(1)
                    wait_scatter(0)
                    sidx_for(j + 1, 1)
                    start_scatter(1)

                    @pl.when(g + 1 < npair)
                    def _():
                        gidx_for(j + 2, 0)
                        start_gather(0)

                    return inner

                lax.fori_loop(0, npair, pair, 0)
                wait_scatter(1)
                return carry

            lax.fori_loop(0, chunk // SB, superblock, 0)

        phase(t1x_h, si_h, di_h, ci, False)
        phase(t1b_h, sb_h, db_h, cb, True)

        plsc.subcore_barrier()

        f0 = s * nf
        pltpu.sync_copy(s_sh.at[pl.ds(f0, nf)], s_out.at[pl.ds(base + f0, nf)])
        pltpu.sync_copy(c_sh.at[pl.ds(f0, nf)], zc.at[pl.ds(0, 624)])
        pltpu.sync_copy(zc.at[pl.ds(0, 624)], c_out.at[pl.ds(base + f0, 624)])
        tail = NTILE * nf
        trem = HALF - tail

        @pl.when(s == 0)
        def _():
            pltpu.sync_copy(s_sh.at[pl.ds(tail, trem)],
                            s_out.at[pl.ds(base + tail, trem)])
            pltpu.sync_copy(c_sh.at[pl.ds(tail, trem)], zc.at[pl.ds(624, trem)])
            pltpu.sync_copy(zc.at[pl.ds(624, trem)],
                            c_out.at[pl.ds(base + tail, trem)])

    return k(t1x, t1b, si, di, sb, db)


def kernel(x_int, bv, edge_index_int, edge_index_bound,
           W_msg, b_msg, W_self, b_self, W_upd, b_upd):
    x = x_int.reshape(-1, D)
    b = bv.reshape(-1, D)
    wa = W_msg[:, :D]
    wb = W_msg[:, D:]
    bm = b_msg.reshape(1, D)
    bs = b_self.reshape(1, D)
    bu = b_upd.reshape(1, D)

    t1x, y2pb, selfx = _mm3(x, wa, wb, W_self, bm, bs, 2000)
    t1b, _, selfb = _mm3(b, wa, wb, W_self, bm, bs, 2000)

    si = edge_index_int[0].astype(jnp.int32)
    di = edge_index_int[1].astype(jnp.int32)
    sb = edge_index_bound[0].astype(jnp.int32)
    db = edge_index_bound[1].astype(jnp.int32)

    S, C = _sc_scatter(t1x, t1b, si, di, sb, db)
    xu = _final(S, C.reshape(10, 1, 2000), y2pb, selfx, W_upd, bu, 2000)
    return xu[None, ...], selfb[None, ...]
